# baseline (speedup 1.0000x reference)
import functools

import jax
import jax.numpy as jnp
from jax import lax
from jax.experimental import pallas as pl
from jax.experimental.pallas import tpu as pltpu
from jax.experimental.pallas import tpu_sc as plsc

N = 10000
E = 320000
HEADS = 4
CH = 32
HC = 128
DE = 16
FC = 256
NG = 64

NC, NS = 2, 16
NW = NC * NS
EPT = E // NW
K = 80
NCHUNK = EPT // K
NP = 10240
RPT = NP // NS


def _proj_body(h_ref, wl_ref, bl_ref, wr_ref, br_ref, xl_ref, xr_ref):
    h = h_ref[...]
    xl_ref[...] = jnp.dot(h, wl_ref[...], preferred_element_type=jnp.float32) + bl_ref[...]
    xr_ref[...] = jnp.dot(h, wr_ref[...], preferred_element_type=jnp.float32) + br_ref[...]


def _proj(h, Wl, bl, Wr, br):
    nb = 2000
    return pl.pallas_call(
        _proj_body,
        grid=(N // nb,),
        in_specs=[
            pl.BlockSpec((nb, HC), lambda i: (i, 0)),
            pl.BlockSpec((HC, HC), lambda i: (0, 0)),
            pl.BlockSpec((1, HC), lambda i: (0, 0)),
            pl.BlockSpec((HC, HC), lambda i: (0, 0)),
            pl.BlockSpec((1, HC), lambda i: (0, 0)),
        ],
        out_specs=[
            pl.BlockSpec((nb, HC), lambda i: (i, 0)),
            pl.BlockSpec((nb, HC), lambda i: (i, 0)),
        ],
        out_shape=[jax.ShapeDtypeStruct((N, HC), jnp.float32)] * 2,
    )(h, Wl, bl.reshape(1, HC), Wr, br.reshape(1, HC))


def _ea_body(a_ref, w0_ref, w1_ref, w2_ref, o0_ref, o1_ref, o2_ref):
    a = a_ref[...]
    o0_ref[...] = jnp.dot(a, w0_ref[...], preferred_element_type=jnp.float32)
    o1_ref[...] = jnp.dot(a, w1_ref[...], preferred_element_type=jnp.float32)
    o2_ref[...] = jnp.dot(a, w2_ref[...], preferred_element_type=jnp.float32)


def _ea_all(edge_attr, W0, W1, W2):
    eb = 8000
    return pl.pallas_call(
        _ea_body,
        grid=(E // eb,),
        in_specs=[
            pl.BlockSpec((eb, DE), lambda i: (i, 0)),
            pl.BlockSpec((DE, HC), lambda i: (0, 0)),
            pl.BlockSpec((DE, HC), lambda i: (0, 0)),
            pl.BlockSpec((DE, HC), lambda i: (0, 0)),
        ],
        out_specs=[pl.BlockSpec((eb, HC), lambda i: (i, 0))] * 3,
        out_shape=[jax.ShapeDtypeStruct((E, HC), jnp.float32)] * 3,
    )(edge_attr, W0, W1, W2)


def _combine_body(p_ref, s_ref, b_ref, h_ref):
    num = p_ref[0] + p_ref[1]
    den = s_ref[0] + s_ref[1]
    h_ref[...] = jnp.maximum(num / (den + 1e-16) + b_ref[...], 0.0)


def _combine(pout, ps, bias):
    nb = 2000
    return pl.pallas_call(
        _combine_body,
        grid=(N // nb,),
        in_specs=[
            pl.BlockSpec((NC, nb, HC), lambda i: (0, i, 0)),
            pl.BlockSpec((NC, nb, HC), lambda i: (0, i, 0)),
            pl.BlockSpec((1, HC), lambda i: (0, 0)),
        ],
        out_specs=pl.BlockSpec((nb, HC), lambda i: (i, 0)),
        out_shape=jax.ShapeDtypeStruct((N, HC), jnp.float32),
    )(pout, ps, bias.reshape(1, HC))


def _bn(h, g_ref, be_ref, rm_ref, rv_ref):
    return ((h - rm_ref[...]) / jnp.sqrt(rv_ref[...] + 1e-5)
            * g_ref[...] + be_ref[...])


def _head_body(h_ref, b3_ref,
               w1_ref, b1_ref, g1_ref, be1_ref, rm1_ref, rv1_ref,
               w2_ref, b2_ref, g2_ref, be2_ref, rm2_ref, rv2_ref,
               w3_ref, bo_ref, out_ref, ssum_ref, cnt_ref):
    i = pl.program_id(0)

    @pl.when(i == 0)
    def _():
        ssum_ref[...] = jnp.zeros_like(ssum_ref)
        cnt_ref[...] = jnp.zeros_like(cnt_ref)

    hb = h_ref[...]
    bblk = b3_ref[0]
    gid = lax.broadcasted_iota(jnp.int32, (NG, 1), 0)
    oh = (bblk == gid).astype(jnp.float32)
    ssum_ref[...] += jnp.dot(oh, hb, preferred_element_type=jnp.float32)
    cnt_ref[...] += jnp.broadcast_to(jnp.sum(oh, axis=1, keepdims=True), (NG, HC))

    @pl.when(i == pl.num_programs(0) - 1)
    def _():
        g = ssum_ref[...] / jnp.maximum(cnt_ref[...], 1.0)
        h1 = jnp.dot(g, w1_ref[...], preferred_element_type=jnp.float32) + b1_ref[...]
        h1 = jnp.maximum(_bn(h1, g1_ref, be1_ref, rm1_ref, rv1_ref), 0.0)
        h2 = jnp.dot(h1, w2_ref[...], preferred_element_type=jnp.float32) + b2_ref[...]
        h2 = jnp.maximum(_bn(h2, g2_ref, be2_ref, rm2_ref, rv2_ref), 0.0)
        out_ref[...] = jnp.dot(h2, w3_ref[...], preferred_element_type=jnp.float32) + bo_ref[...]


def _head(h, batch, p):
    nb = 1000
    nblk = N // nb
    b3 = batch.reshape(nblk, 1, nb)
    full = lambda shape: pl.BlockSpec(shape, lambda i: tuple(0 for _ in shape))
    row = lambda d: full((1, d))
    return pl.pallas_call(
        _head_body,
        grid=(nblk,),
        in_specs=[
            pl.BlockSpec((nb, HC), lambda i: (i, 0)),
            pl.BlockSpec((1, 1, nb), lambda i: (i, 0, 0)),
            full((HC, FC)), row(FC), row(FC), row(FC), row(FC), row(FC),
            full((FC, FC)), row(FC), row(FC), row(FC), row(FC), row(FC),
            full((FC, 2)), row(2),
        ],
        out_specs=full((NG, 2)),
        out_shape=jax.ShapeDtypeStruct((NG, 2), jnp.float32),
        scratch_shapes=[
            pltpu.VMEM((NG, HC), jnp.float32),
            pltpu.VMEM((NG, HC), jnp.float32),
        ],
    )(h, b3,
      p['W1'], p['b1'].reshape(1, FC), p['g1'].reshape(1, FC),
      p['be1'].reshape(1, FC), p['rm1'].reshape(1, FC), p['rv1'].reshape(1, FC),
      p['W2'], p['b2'].reshape(1, FC), p['g2'].reshape(1, FC),
      p['be2'].reshape(1, FC), p['rm2'].reshape(1, FC), p['rv2'].reshape(1, FC),
      p['W3'], p['b3'].reshape(1, 2))


@functools.cache
def _build_edge_sc():
    mesh = plsc.VectorSubcoreMesh(core_axis_name="c", subcore_axis_name="s",
                                  num_cores=NC, num_subcores=NS)
    return functools.partial(
        pl.kernel,
        out_type=[jax.ShapeDtypeStruct((NC, NP, HC), jnp.float32),
                  jax.ShapeDtypeStruct((NC, NP, HC), jnp.float32)],
        mesh=mesh,
        compiler_params=pltpu.CompilerParams(needs_layout_passes=False,
                                             use_tc_tiling_on_sc=False),
        scratch_types=[
            pltpu.VMEM_SHARED((NP, HC), jnp.float32),
            pltpu.VMEM_SHARED((NP, 16), jnp.float32),
            pltpu.VMEM((K,), jnp.int32),
            pltpu.VMEM((K,), jnp.int32),
            pltpu.VMEM((K, HC), jnp.float32),
            pltpu.VMEM((K, HC), jnp.float32),
            pltpu.VMEM((K, HC), jnp.float32),
            pltpu.VMEM((K, 16), jnp.float32),
            pltpu.VMEM((HC,), jnp.float32),
            pltpu.VMEM((HEADS, 16, 16), jnp.float32),
            pltpu.VMEM((HEADS, 16), jnp.float32),
            pltpu.SemaphoreType.DMA,
            pltpu.SemaphoreType.DMA,
        ],
    )(_edge_sc_body)


def _edge_sc(xl, xr, ea, src, dst, att128):
    return _build_edge_sc()(xl, xr, ea, src, dst, att128)


def _edge_sc_body(xl_hbm, xr_hbm, ea_hbm, src_hbm, dst_hbm, att_hbm,
                  outp_hbm, outs_hbm,
                  acc, sacc, src_v, dst_v, xl_b, xr_b, ea_b, ex_b,
                  att_v, q_buf, exp_buf, sem1, sem2):
    c = lax.axis_index("c")
    s = lax.axis_index("s")
    wid = s * NC + c
    row0 = s * RPT

    pltpu.sync_copy(att_hbm, att_v)

    zero16 = jnp.zeros((16,), jnp.float32)
    lane = lax.iota(jnp.int32, 16)
    lane0 = lane * 0
    lane3 = lane & 3

    @pl.loop(0, K)
    def _zfill(r):
        for g in range(8):
            ea_b[r, pl.ds(g * 16, 16)] = zero16
        ex_b[r, :] = zero16

    for i in range(RPT // K):
        pltpu.sync_copy(ea_b, acc.at[pl.ds(row0 + i * K, K)])
    for i in range(RPT // K):
        pltpu.sync_copy(ex_b, sacc.at[pl.ds(row0 + i * K, K)])
    plsc.subcore_barrier()

    att_regs = [att_v[pl.ds(g * 16, 16)] for g in range(8)]
    e_base = wid * EPT

    @pl.loop(0, NCHUNK)
    def _chunk(ci):
        e0 = e_base + ci * K
        pltpu.sync_copy(src_hbm.at[pl.ds(e0, K)], src_v)
        pltpu.sync_copy(dst_hbm.at[pl.ds(e0, K)], dst_v)
        cp1 = pltpu.async_copy(xl_hbm.at[src_v], xl_b, sem1)
        cp2 = pltpu.async_copy(xr_hbm.at[dst_v], xr_b, sem2)
        pltpu.sync_copy(ea_hbm.at[pl.ds(e0, K)], ea_b)
        cp1.wait()
        cp2.wait()

        @pl.loop(0, K // 16)
        def _batch(bb):
            j0 = bb * 16
            for dj in range(16):
                j = j0 + dj
                for h in range(HEADS):
                    pv = []
                    for g in (2 * h, 2 * h + 1):
                        z = (xl_b[j, pl.ds(g * 16, 16)]
                             + xr_b[j, pl.ds(g * 16, 16)]
                             + ea_b[j, pl.ds(g * 16, 16)])
                        z = jnp.maximum(z, 0.0) + 0.2 * jnp.minimum(z, 0.0)
                        pv.append(z * att_regs[g])
                    q_buf[h, dj, :] = pv[0] + pv[1]

            for h in range(HEADS):
                hrow = lane0 + h
                parts = []
                for c0 in range(0, 16, 4):
                    t01 = (plsc.load_gather(q_buf, [hrow, lane, lane0 + c0])
                           + plsc.load_gather(q_buf, [hrow, lane, lane0 + c0 + 1]))
                    t23 = (plsc.load_gather(q_buf, [hrow, lane, lane0 + c0 + 2])
                           + plsc.load_gather(q_buf, [hrow, lane, lane0 + c0 + 3]))
                    parts.append(t01 + t23)
                s = (parts[0] + parts[1]) + (parts[2] + parts[3])
                exp_buf[h, :] = jnp.exp(jnp.minimum(s, 60.0))

            for dj in range(16):
                j = j0 + dj
                djv = lane0 + dj
                ex_b[j, :] = plsc.load_gather(exp_buf, [lane3, djv])
                for h in range(HEADS):
                    exv = plsc.load_gather(exp_buf, [lane0 + h, djv])
                    ea_b[j, pl.ds((2 * h) * 16, 16)] = (
                        xl_b[j, pl.ds((2 * h) * 16, 16)] * exv)
                    ea_b[j, pl.ds((2 * h + 1) * 16, 16)] = (
                        xl_b[j, pl.ds((2 * h + 1) * 16, 16)] * exv)

        pltpu.sync_copy(ea_b, acc.at[dst_v], add=True)
        pltpu.sync_copy(ex_b, sacc.at[dst_v], add=True)

    plsc.subcore_barrier()

    for i in range(RPT // K):
        r0 = row0 + i * K
        pltpu.sync_copy(acc.at[pl.ds(r0, K)], outp_hbm.at[c, pl.ds(r0, K)])
        pltpu.sync_copy(sacc.at[pl.ds(r0, K)], ex_b)

        @pl.loop(0, K)
        def _widen(j):
            for h in range(HEADS):
                dv = plsc.load_gather(ex_b, [lane0 + j, lane0 + h])
                xl_b[j, pl.ds((2 * h) * 16, 16)] = dv
                xl_b[j, pl.ds((2 * h + 1) * 16, 16)] = dv

        pltpu.sync_copy(xl_b, outs_hbm.at[c, pl.ds(r0, K)])


def kernel(x, edge_index, edge_attr, batch, params):
    src = edge_index[0]
    dst = edge_index[1]
    gats = params['gats']
    eas = _ea_all(edge_attr, gats[0]['We'], gats[1]['We'], gats[2]['We'])
    h = x
    for l in range(3):
        p = gats[l]
        xl, xr = _proj(h, p['Wl'], p['bl'], p['Wr'], p['br'])
        pout, ps = _edge_sc(xl, xr, eas[l], src, dst, p['att'].reshape(HC))
        h = _combine(pout, ps, p['bias'])
    return _head(h, batch, params)


# speedup vs baseline: 1.7635x; 1.7635x over previous
import functools

import jax
import jax.numpy as jnp
from jax import lax
from jax.experimental import pallas as pl
from jax.experimental.pallas import tpu as pltpu
from jax.experimental.pallas import tpu_sc as plsc

N = 10000
E = 320000
HEADS = 4
CH = 32
HC = 128
DE = 16
FC = 256
NG = 64

NC, NS = 2, 16
NW = NC * NS
EPT = E // NW
K = 40
NCHUNK = EPT // K
NP = 10240
RPT = NP // NS


def _proj_body(h_ref, wl_ref, bl_ref, wr_ref, br_ref, xl_ref, xr_ref):
    h = h_ref[...]
    xl_ref[...] = jnp.dot(h, wl_ref[...], preferred_element_type=jnp.float32) + bl_ref[...]
    xr_ref[...] = jnp.dot(h, wr_ref[...], preferred_element_type=jnp.float32) + br_ref[...]


def _proj(h, Wl, bl, Wr, br):
    nb = 2000
    return pl.pallas_call(
        _proj_body,
        grid=(N // nb,),
        in_specs=[
            pl.BlockSpec((nb, HC), lambda i: (i, 0)),
            pl.BlockSpec((HC, HC), lambda i: (0, 0)),
            pl.BlockSpec((1, HC), lambda i: (0, 0)),
            pl.BlockSpec((HC, HC), lambda i: (0, 0)),
            pl.BlockSpec((1, HC), lambda i: (0, 0)),
        ],
        out_specs=[
            pl.BlockSpec((nb, HC), lambda i: (i, 0)),
            pl.BlockSpec((nb, HC), lambda i: (i, 0)),
        ],
        out_shape=[jax.ShapeDtypeStruct((N, HC), jnp.float32)] * 2,
    )(h, Wl, bl.reshape(1, HC), Wr, br.reshape(1, HC))


def _ea_body(a_ref, w0_ref, w1_ref, w2_ref, o0_ref, o1_ref, o2_ref):
    a = a_ref[...]
    o0_ref[...] = jnp.dot(a, w0_ref[...], preferred_element_type=jnp.float32)
    o1_ref[...] = jnp.dot(a, w1_ref[...], preferred_element_type=jnp.float32)
    o2_ref[...] = jnp.dot(a, w2_ref[...], preferred_element_type=jnp.float32)


def _ea_all(edge_attr, W0, W1, W2):
    eb = 8000
    return pl.pallas_call(
        _ea_body,
        grid=(E // eb,),
        in_specs=[
            pl.BlockSpec((eb, DE), lambda i: (i, 0)),
            pl.BlockSpec((DE, HC), lambda i: (0, 0)),
            pl.BlockSpec((DE, HC), lambda i: (0, 0)),
            pl.BlockSpec((DE, HC), lambda i: (0, 0)),
        ],
        out_specs=[pl.BlockSpec((eb, HC), lambda i: (i, 0))] * 3,
        out_shape=[jax.ShapeDtypeStruct((E, HC), jnp.float32)] * 3,
    )(edge_attr, W0, W1, W2)


def _combine_body(p_ref, s_ref, b_ref, h_ref):
    num = p_ref[0] + p_ref[1]
    den = s_ref[0] + s_ref[1]
    h_ref[...] = jnp.maximum(num / (den + 1e-16) + b_ref[...], 0.0)


def _combine(pout, ps, bias):
    nb = 2000
    return pl.pallas_call(
        _combine_body,
        grid=(N // nb,),
        in_specs=[
            pl.BlockSpec((NC, nb, HC), lambda i: (0, i, 0)),
            pl.BlockSpec((NC, nb, HC), lambda i: (0, i, 0)),
            pl.BlockSpec((1, HC), lambda i: (0, 0)),
        ],
        out_specs=pl.BlockSpec((nb, HC), lambda i: (i, 0)),
        out_shape=jax.ShapeDtypeStruct((N, HC), jnp.float32),
    )(pout, ps, bias.reshape(1, HC))


def _bn(h, g_ref, be_ref, rm_ref, rv_ref):
    return ((h - rm_ref[...]) / jnp.sqrt(rv_ref[...] + 1e-5)
            * g_ref[...] + be_ref[...])


def _head_body(h_ref, b3_ref,
               w1_ref, b1_ref, g1_ref, be1_ref, rm1_ref, rv1_ref,
               w2_ref, b2_ref, g2_ref, be2_ref, rm2_ref, rv2_ref,
               w3_ref, bo_ref, out_ref, ssum_ref, cnt_ref):
    i = pl.program_id(0)

    @pl.when(i == 0)
    def _():
        ssum_ref[...] = jnp.zeros_like(ssum_ref)
        cnt_ref[...] = jnp.zeros_like(cnt_ref)

    hb = h_ref[...]
    bblk = b3_ref[0]
    gid = lax.broadcasted_iota(jnp.int32, (NG, 1), 0)
    oh = (bblk == gid).astype(jnp.float32)
    ssum_ref[...] += jnp.dot(oh, hb, preferred_element_type=jnp.float32)
    cnt_ref[...] += jnp.broadcast_to(jnp.sum(oh, axis=1, keepdims=True), (NG, HC))

    @pl.when(i == pl.num_programs(0) - 1)
    def _():
        g = ssum_ref[...] / jnp.maximum(cnt_ref[...], 1.0)
        h1 = jnp.dot(g, w1_ref[...], preferred_element_type=jnp.float32) + b1_ref[...]
        h1 = jnp.maximum(_bn(h1, g1_ref, be1_ref, rm1_ref, rv1_ref), 0.0)
        h2 = jnp.dot(h1, w2_ref[...], preferred_element_type=jnp.float32) + b2_ref[...]
        h2 = jnp.maximum(_bn(h2, g2_ref, be2_ref, rm2_ref, rv2_ref), 0.0)
        out_ref[...] = jnp.dot(h2, w3_ref[...], preferred_element_type=jnp.float32) + bo_ref[...]


def _head(h, batch, p):
    nb = 1000
    nblk = N // nb
    b3 = batch.reshape(nblk, 1, nb)
    full = lambda shape: pl.BlockSpec(shape, lambda i: tuple(0 for _ in shape))
    row = lambda d: full((1, d))
    return pl.pallas_call(
        _head_body,
        grid=(nblk,),
        in_specs=[
            pl.BlockSpec((nb, HC), lambda i: (i, 0)),
            pl.BlockSpec((1, 1, nb), lambda i: (i, 0, 0)),
            full((HC, FC)), row(FC), row(FC), row(FC), row(FC), row(FC),
            full((FC, FC)), row(FC), row(FC), row(FC), row(FC), row(FC),
            full((FC, 2)), row(2),
        ],
        out_specs=full((NG, 2)),
        out_shape=jax.ShapeDtypeStruct((NG, 2), jnp.float32),
        scratch_shapes=[
            pltpu.VMEM((NG, HC), jnp.float32),
            pltpu.VMEM((NG, HC), jnp.float32),
        ],
    )(h, b3,
      p['W1'], p['b1'].reshape(1, FC), p['g1'].reshape(1, FC),
      p['be1'].reshape(1, FC), p['rm1'].reshape(1, FC), p['rv1'].reshape(1, FC),
      p['W2'], p['b2'].reshape(1, FC), p['g2'].reshape(1, FC),
      p['be2'].reshape(1, FC), p['rm2'].reshape(1, FC), p['rv2'].reshape(1, FC),
      p['W3'], p['b3'].reshape(1, 2))


@functools.cache
def _build_edge_sc():
    mesh = plsc.VectorSubcoreMesh(core_axis_name="c", subcore_axis_name="s",
                                  num_cores=NC, num_subcores=NS)
    return functools.partial(
        pl.kernel,
        out_type=[jax.ShapeDtypeStruct((NC, NP, HC), jnp.float32),
                  jax.ShapeDtypeStruct((NC, NP, HC), jnp.float32)],
        mesh=mesh,
        compiler_params=pltpu.CompilerParams(needs_layout_passes=False,
                                             use_tc_tiling_on_sc=False),
        scratch_types=[
            pltpu.VMEM_SHARED((NP, HC), jnp.float32),
            pltpu.VMEM_SHARED((NP, 16), jnp.float32),
            pltpu.VMEM((K,), jnp.int32),
            pltpu.VMEM((K,), jnp.int32),
            pltpu.VMEM((K, HC), jnp.float32),
            pltpu.VMEM((K, HC), jnp.float32),
            pltpu.VMEM((K, HC), jnp.float32),
            pltpu.VMEM((K,), jnp.int32),
            pltpu.VMEM((K,), jnp.int32),
            pltpu.VMEM((K, HC), jnp.float32),
            pltpu.VMEM((K, HC), jnp.float32),
            pltpu.VMEM((K, HC), jnp.float32),
            pltpu.VMEM((K, 16), jnp.float32),
            pltpu.VMEM((HC,), jnp.float32),
            pltpu.VMEM((32, 16), jnp.float32),
            pltpu.SemaphoreType.DMA,
            pltpu.SemaphoreType.DMA,
            pltpu.SemaphoreType.DMA,
            pltpu.SemaphoreType.DMA,
            pltpu.SemaphoreType.DMA,
            pltpu.SemaphoreType.DMA,
        ],
    )(_edge_sc_body)


def _edge_sc(xl, xr, ea, src, dst, att128):
    return _build_edge_sc()(xl, xr, ea, src, dst, att128)


def _edge_sc_body(xl_hbm, xr_hbm, ea_hbm, src_hbm, dst_hbm, att_hbm,
                  outp_hbm, outs_hbm,
                  acc, sacc,
                  src_a, dst_a, xl_a, xr_a, ea_a,
                  src_b, dst_b, xl_b2, xr_b2, ea_b2,
                  ex_b, att_v, red_v,
                  sxa, sra, sea, sxb, srb, seb):
    bufs = ((src_a, dst_a, xl_a, xr_a, ea_a, sxa, sra, sea),
            (src_b, dst_b, xl_b2, xr_b2, ea_b2, sxb, srb, seb))
    src_v, dst_v, xl_b, xr_b, ea_b = src_a, dst_a, xl_a, xr_a, ea_a
    c = lax.axis_index("c")
    s = lax.axis_index("s")
    wid = s * NC + c
    row0 = s * RPT

    pltpu.sync_copy(att_hbm, att_v)

    zero16 = jnp.zeros((16,), jnp.float32)
    lane = lax.iota(jnp.int32, 16)
    lane0 = lane * 0
    xor1 = lane ^ 1
    xor2 = lane ^ 2
    packr = lax.shift_right_logical(lane, 2)
    packc = (lane & 3) * 4

    def head_reduce(qs, p):
        base = p * 8
        for h in range(HEADS):
            r = lane0 + (base + h)
            red_v[base + h, :] = qs[h]
            t = qs[h] + plsc.load_gather(red_v, [r, xor1])
            red_v[base + h, :] = t
            u = t + plsc.load_gather(red_v, [r, xor2])
            red_v[base + h, :] = u
        m = plsc.load_gather(red_v, [packr + base, packc])
        red_v[base + 4, :] = m
        m = m + plsc.load_gather(red_v, [lane0 + (base + 4), xor1])
        red_v[base + 5, :] = m
        m = m + plsc.load_gather(red_v, [lane0 + (base + 5), xor2])
        ex_m = jnp.exp(jnp.minimum(m, 60.0))
        red_v[base + 6, :] = ex_m
        r6 = lane0 + (base + 6)
        exvs = [plsc.load_gather(red_v, [r6, lane0 + 4 * h])
                for h in range(HEADS)]
        exrow = plsc.load_gather(red_v, [r6, packc])
        return exvs, exrow

    @pl.loop(0, K)
    def _zfill(r):
        for g in range(8):
            ea_b[r, pl.ds(g * 16, 16)] = zero16
        ex_b[r, :] = zero16

    for i in range(RPT // K):
        pltpu.sync_copy(ea_b, acc.at[pl.ds(row0 + i * K, K)])
    for i in range(RPT // K):
        pltpu.sync_copy(ex_b, sacc.at[pl.ds(row0 + i * K, K)])
    plsc.subcore_barrier()

    att_regs = [att_v[pl.ds(g * 16, 16)] for g in range(8)]
    e_base = wid * EPT

    def issue(ci, bset):
        sv, dv, xb, rb, eb, s1, s2, s3 = bset
        e0 = e_base + ci * K
        pltpu.sync_copy(src_hbm.at[pl.ds(e0, K)], sv)
        pltpu.sync_copy(dst_hbm.at[pl.ds(e0, K)], dv)
        pltpu.async_copy(xl_hbm.at[sv], xb, s1)
        pltpu.async_copy(xr_hbm.at[dv], rb, s2)
        pltpu.async_copy(ea_hbm.at[pl.ds(e0, K)], eb, s3)

    def wait(bset):
        sv, dv, xb, rb, eb, s1, s2, s3 = bset
        pltpu.make_async_copy(xl_hbm.at[sv], xb, s1).wait()
        pltpu.make_async_copy(xr_hbm.at[dv], rb, s2).wait()
        pltpu.make_async_copy(ea_hbm.at[pl.ds(e_base, K)], eb, s3).wait()

    def compute(bset):
        sv, dv, xb, rb, eb, s1, s2, s3 = bset

        def _one_edge(j, p):
            zl = [xb[j, pl.ds(g * 16, 16)] for g in range(8)]
            qs = []
            for h in range(HEADS):
                pv = []
                for g in (2 * h, 2 * h + 1):
                    z = (zl[g] + rb[j, pl.ds(g * 16, 16)]
                         + eb[j, pl.ds(g * 16, 16)])
                    z = jnp.maximum(z, 0.0) + 0.2 * jnp.minimum(z, 0.0)
                    pv.append(z * att_regs[g])
                qs.append(pv[0] + pv[1])
            exvs, exrow = head_reduce(qs, p)
            for h in range(HEADS):
                eb[j, pl.ds((2 * h) * 16, 16)] = zl[2 * h] * exvs[h]
                eb[j, pl.ds((2 * h + 1) * 16, 16)] = zl[2 * h + 1] * exvs[h]
            ex_b[j, :] = exrow

        @pl.loop(0, K // 2)
        def _edge(jj):
            j = jj * 2
            _one_edge(j, 0)
            _one_edge(j + 1, 1)

        pltpu.sync_copy(eb, acc.at[dv], add=True)
        pltpu.sync_copy(ex_b, sacc.at[dv], add=True)

    issue(0, bufs[0])
    issue(1, bufs[1])

    @pl.loop(0, NCHUNK // 2)
    def _chunk(ci):
        c0 = 2 * ci
        wait(bufs[0])
        compute(bufs[0])

        @pl.when(c0 + 2 < NCHUNK)
        def _():
            issue(c0 + 2, bufs[0])

        wait(bufs[1])
        compute(bufs[1])

        @pl.when(c0 + 3 < NCHUNK)
        def _():
            issue(c0 + 3, bufs[1])

    plsc.subcore_barrier()

    for i in range(RPT // K):
        r0 = row0 + i * K
        pltpu.sync_copy(acc.at[pl.ds(r0, K)], outp_hbm.at[c, pl.ds(r0, K)])
        pltpu.sync_copy(sacc.at[pl.ds(r0, K)], ex_b)

        @pl.loop(0, K)
        def _widen(j):
            for h in range(HEADS):
                dv = plsc.load_gather(ex_b, [lane0 + j, lane0 + h])
                xl_b[j, pl.ds((2 * h) * 16, 16)] = dv
                xl_b[j, pl.ds((2 * h + 1) * 16, 16)] = dv

        pltpu.sync_copy(xl_b, outs_hbm.at[c, pl.ds(r0, K)])


def kernel(x, edge_index, edge_attr, batch, params):
    src = edge_index[0]
    dst = edge_index[1]
    gats = params['gats']
    eas = _ea_all(edge_attr, gats[0]['We'], gats[1]['We'], gats[2]['We'])
    h = x
    for l in range(3):
        p = gats[l]
        xl, xr = _proj(h, p['Wl'], p['bl'], p['Wr'], p['br'])
        pout, ps = _edge_sc(xl, xr, eas[l], src, dst, p['att'].reshape(HC))
        h = _combine(pout, ps, p['bias'])
    return _head(h, batch, params)
